# Initial kernel scaffold; baseline (speedup 1.0000x reference)
import jax
import jax.numpy as jnp
from jax.experimental import pallas as pl


def kernel(x, adjacency):
    raise NotImplementedError("write your pallas kernel here")


# speedup vs baseline: 2.0253x; 2.0253x over previous
import jax
import jax.numpy as jnp
from jax.experimental import pallas as pl


def _fused_body(x_ref, a_ref, o_ref):
    a = a_ref[...]
    support = jnp.dot(a, x_ref[...], preferred_element_type=jnp.float32)
    num_neigh = jnp.sum(a, axis=1, keepdims=True)
    o_ref[...] = support / num_neigh


def kernel(x, adjacency):
    n, d = x.shape
    bi = 256
    grid = (n // bi,)
    return pl.pallas_call(
        _fused_body,
        grid=grid,
        in_specs=[
            pl.BlockSpec((n, d), lambda i: (0, 0)),
            pl.BlockSpec((bi, n), lambda i: (i, 0)),
        ],
        out_specs=pl.BlockSpec((bi, d), lambda i: (i, 0)),
        out_shape=jax.ShapeDtypeStruct((n, d), jnp.float32),
    )(x, adjacency)
